# Initial kernel scaffold; baseline (speedup 1.0000x reference)
import jax
import jax.numpy as jnp
from jax.experimental import pallas as pl


def kernel(x, table, V_w, V_b, W_w, W_b):
    raise NotImplementedError("write your pallas kernel here")


# speedup vs baseline: 2.1423x; 2.1423x over previous
import functools

import jax
import jax.numpy as jnp
from jax import lax
from jax.experimental import pallas as pl
from jax.experimental.pallas import tpu as pltpu
from jax.experimental.pallas import tpu_sc as plsc

_VOCAB = 1000000
_D = 64
_HIST = 50
_BATCH = 16384
_HID = 256
_NUM_CLASSES = 2

_NC = 2
_NS = 16
_NW = _NC * _NS
_ROWS_PER_W = _BATCH // _NW
_CHUNK_ROWS = 2
_CHUNK_IDX = _CHUNK_ROWS * _HIST
_NCHUNK = _ROWS_PER_W // _CHUNK_ROWS
_NBUF = 2


def _sc_gather_mean(table, x_chunks):
  mesh = plsc.VectorSubcoreMesh(
      core_axis_name="c", subcore_axis_name="s",
      num_cores=_NC, num_subcores=_NS)

  @functools.partial(
      pl.kernel,
      out_type=jax.ShapeDtypeStruct((_BATCH, _D), jnp.float32),
      mesh=mesh,
      scratch_types=[
          pltpu.VMEM((_NCHUNK, _CHUNK_IDX), jnp.int32),
          pltpu.VMEM((_NBUF, _CHUNK_IDX, _D), jnp.float32),
          pltpu.VMEM((_ROWS_PER_W, _D), jnp.float32),
          pltpu.SemaphoreType.DMA((_NBUF,)),
      ],
      compiler_params=pltpu.CompilerParams(use_tc_tiling_on_sc=False),
  )
  def sc_kernel(table_hbm, x_hbm, out_hbm, idx_v, buf_v, out_v, sems):
    wid = lax.axis_index("c") * _NS + lax.axis_index("s")
    chunk_base = wid * _NCHUNK
    pltpu.sync_copy(x_hbm.at[pl.ds(chunk_base, _NCHUNK)], idx_v)

    def issue(j, b):
      pltpu.async_copy(table_hbm.at[idx_v.at[j]], buf_v.at[b], sems.at[b])

    for b in range(_NBUF):
      issue(b, b)

    def drain_wait(b):
      pltpu.make_async_copy(table_hbm.at[idx_v.at[0]], buf_v.at[b],
                            sems.at[b]).wait()

    def body(j):
      for b in range(_NBUF):
        jj = j + b
        drain_wait(b)
        for r2 in range(_CHUNK_ROWS):
          for s in range(_D // 16):
            sl = pl.ds(s * 16, 16)
            acc = buf_v[b, r2 * _HIST, sl]
            for r in range(1, _HIST):
              acc = acc + buf_v[b, r2 * _HIST + r, sl]
            out_v[jj * _CHUNK_ROWS + r2, sl] = acc * (1.0 / _HIST)
        nxt = jj + _NBUF
        @pl.when(nxt < _NCHUNK)
        def _():
          issue(nxt, b)

    lax.fori_loop(0, _NCHUNK // _NBUF, lambda i, _: (body(i * _NBUF), 0)[1], 0,
                  unroll=False)
    pltpu.sync_copy(out_v, out_hbm.at[pl.ds(wid * _ROWS_PER_W, _ROWS_PER_W)])

  return sc_kernel(table, x_chunks)


def _tc_mlp(averaged, V_w, V_b, W_wp, W_bp):
  blk = 1024

  def mlp_kernel(avg_ref, vw_ref, vb_ref, ww_ref, wb_ref, out_ref):
    h = jnp.dot(avg_ref[...], vw_ref[...], preferred_element_type=jnp.float32)
    h = jnp.maximum(h + vb_ref[...], 0.0)
    logits = jnp.dot(h, ww_ref[...], preferred_element_type=jnp.float32)
    logits = logits + wb_ref[...]
    col = lax.broadcasted_iota(jnp.int32, logits.shape, 1)
    valid = col < _NUM_CLASSES
    neg = jnp.full_like(logits, -jnp.inf)
    masked = jnp.where(valid, logits, neg)
    m = jnp.max(masked, axis=1, keepdims=True)
    ex = jnp.where(valid, jnp.exp(masked - m), 0.0)
    lse = jnp.log(jnp.sum(ex, axis=1, keepdims=True)) + m
    out_ref[...] = logits - lse

  grid = _BATCH // blk
  return pl.pallas_call(
      mlp_kernel,
      grid=(grid,),
      in_specs=[
          pl.BlockSpec((blk, _D), lambda i: (i, 0)),
          pl.BlockSpec((_D, _HID), lambda i: (0, 0)),
          pl.BlockSpec((1, _HID), lambda i: (0, 0)),
          pl.BlockSpec((_HID, 128), lambda i: (0, 0)),
          pl.BlockSpec((1, 128), lambda i: (0, 0)),
      ],
      out_specs=pl.BlockSpec((blk, 128), lambda i: (i, 0)),
      out_shape=jax.ShapeDtypeStruct((_BATCH, 128), jnp.float32),
  )(averaged, V_w, V_b, W_wp, W_bp)


def kernel(x, table, V_w, V_b, W_w, W_b):
  x_chunks = x.reshape(_NW * _NCHUNK, _CHUNK_IDX)
  averaged = _sc_gather_mean(table, x_chunks)
  W_wp = jnp.pad(W_w, ((0, 0), (0, 128 - _NUM_CLASSES)))
  W_bp = jnp.pad(W_b, (0, 128 - _NUM_CLASSES)).reshape(1, 128)
  out_full = _tc_mlp(averaged, V_w, V_b.reshape(1, _HID), W_wp, W_bp)
  return out_full[:, :_NUM_CLASSES]
